# baseline (speedup 1.0000x reference)
import functools

import jax
import jax.numpy as jnp
from jax import lax
from jax.experimental import pallas as pl
from jax.experimental.pallas import tpu as pltpu
from jax.experimental.pallas import tpu_sc as plsc

N_NODES = 100000
N_EDGES = 3200000
NUM_GRAPHS = 64

NC = 2
NS = 16
NW = NC * NS

CH = 128
KJ = 16
SB = CH * KJ
NB = -(-N_EDGES // (NW * SB))
E_PAD = NW * NB * SB

N_PAD = 100352
TSLICE = N_PAD // NS
Q2 = 8
Q1 = 8
NBLK = 2048
NGRID = N_PAD // NBLK

_MESH = plsc.VectorSubcoreMesh(core_axis_name="c", subcore_axis_name="s")


@functools.partial(
    pl.kernel,
    out_type=jax.ShapeDtypeStruct((NC, N_PAD), jnp.float32),
    mesh=_MESH,
    compiler_params=pltpu.CompilerParams(use_tc_tiling_on_sc=False),
    scratch_types=[
        pltpu.VMEM((KJ, CH), jnp.int32),
        pltpu.VMEM((KJ, CH), jnp.float32),
        pltpu.VMEM_SHARED((N_PAD,), jnp.float32),
        pltpu.SemaphoreType.DMA,
    ],
)
def _deg_kernel(col_hbm, ew_hbm, zeros_hbm, out_hbm, idx_v, val_v, acc_sh,
                sem):
    c = lax.axis_index("c")
    s = lax.axis_index("s")
    wid = s * NC + c
    sl = pl.ds(s * TSLICE, TSLICE)
    pltpu.sync_copy(zeros_hbm.at[sl], acc_sh.at[sl])
    plsc.subcore_barrier()

    def blk(b, carry):
        chunk0 = (wid * NB + b) * KJ
        pltpu.sync_copy(col_hbm.at[pl.ds(chunk0, KJ)], idx_v)
        pltpu.sync_copy(ew_hbm.at[pl.ds(chunk0, KJ)], val_v)
        descs = [
            pltpu.async_copy(val_v.at[j], acc_sh.at[idx_v.at[j]], sem,
                             add=True)
            for j in range(KJ)
        ]
        for d in descs:
            d.wait()
        return carry

    lax.fori_loop(0, NB, blk, 0)
    plsc.subcore_barrier()
    pltpu.sync_copy(acc_sh.at[sl], out_hbm.at[c, sl])


@functools.partial(
    pl.kernel,
    out_type=jax.ShapeDtypeStruct((NC * 3, N_PAD), jnp.float32),
    mesh=_MESH,
    compiler_params=pltpu.CompilerParams(use_tc_tiling_on_sc=False),
    scratch_types=[
        pltpu.VMEM((KJ, CH), jnp.int32),
        pltpu.VMEM((KJ, CH), jnp.int32),
        pltpu.VMEM((KJ, CH), jnp.float32),
        pltpu.VMEM((Q1 * 3, CH), jnp.float32),
        pltpu.VMEM_SHARED((N_PAD,), jnp.float32),
        pltpu.VMEM_SHARED((N_PAD,), jnp.float32),
        pltpu.VMEM_SHARED((N_PAD,), jnp.float32),
        pltpu.SemaphoreType.DMA,
        pltpu.SemaphoreType.DMA,
    ],
)
def _agg1_kernel(row_hbm, col_hbm, ew_hbm, t0, t1, t2, zeros_hbm, out_hbm,
                 idxr_v, idxc_v, ew_v, gbuf_v, acc0, acc1, acc2, gsem, ssem):
    c = lax.axis_index("c")
    s = lax.axis_index("s")
    wid = s * NC + c
    sl = pl.ds(s * TSLICE, TSLICE)
    accs = [acc0, acc1, acc2]
    tabs = [t0, t1, t2]
    for f in range(3):
        pltpu.sync_copy(zeros_hbm.at[sl], accs[f].at[sl])
    plsc.subcore_barrier()

    def blk(b, carry):
        chunk0 = (wid * NB + b) * KJ
        pltpu.sync_copy(row_hbm.at[pl.ds(chunk0, KJ)], idxr_v)
        pltpu.sync_copy(col_hbm.at[pl.ds(chunk0, KJ)], idxc_v)
        pltpu.sync_copy(ew_hbm.at[pl.ds(chunk0, KJ)], ew_v)

        def qloop(q, carry2):
            j0 = q * Q1
            gds = [
                pltpu.async_copy(tabs[f].at[idxr_v.at[j0 + jj]],
                                 gbuf_v.at[jj * 3 + f], gsem)
                for jj in range(Q1) for f in range(3)
            ]
            for d in gds:
                d.wait()
            for jj in range(Q1):
                for f in range(3):
                    for g in range(CH // 16):
                        qq = pl.ds(g * 16, 16)
                        gbuf_v[jj * 3 + f, qq] = \
                            gbuf_v[jj * 3 + f, qq] * ew_v[j0 + jj, qq]
            sds = [
                pltpu.async_copy(gbuf_v.at[jj * 3 + f],
                                 accs[f].at[idxc_v.at[j0 + jj]], ssem,
                                 add=True)
                for jj in range(Q1) for f in range(3)
            ]
            for d in sds:
                d.wait()
            return carry2

        lax.fori_loop(0, KJ // Q1, qloop, 0)
        return carry

    lax.fori_loop(0, NB, blk, 0)
    plsc.subcore_barrier()
    for f in range(3):
        pltpu.sync_copy(accs[f].at[sl], out_hbm.at[c * 3 + f, sl])


@functools.partial(
    pl.kernel,
    out_type=jax.ShapeDtypeStruct((NC, N_PAD, 16), jnp.float32),
    mesh=_MESH,
    compiler_params=pltpu.CompilerParams(use_tc_tiling_on_sc=False),
    scratch_types=[
        pltpu.VMEM((KJ, CH), jnp.int32),
        pltpu.VMEM((KJ, CH), jnp.int32),
        pltpu.VMEM((KJ, CH), jnp.float32),
        pltpu.VMEM((4, CH, 16), jnp.float32),
        pltpu.VMEM_SHARED((N_PAD, 16), jnp.float32),
        [pltpu.SemaphoreType.DMA] * 4,
        [pltpu.SemaphoreType.DMA] * 4,
    ],
)
def _agg2_kernel(row_hbm, col_hbm, ew_hbm, tab_hbm, zeros16_hbm, out_hbm,
                 idxr_v, idxc_v, ew_v, rows_v, acc_sh, gsems, ssems):
    c = lax.axis_index("c")
    s = lax.axis_index("s")
    wid = s * NC + c
    sl = pl.ds(s * TSLICE, TSLICE)
    pltpu.sync_copy(zeros16_hbm.at[sl], acc_sh.at[sl])
    plsc.subcore_barrier()

    def scale(j, i):
        for g in range(CH // 16):
            wv = ew_v[j, pl.ds(g * 16, 16)]
            for l in range(16):
                e = g * 16 + l
                rows_v[i, e, :] = rows_v[i, e, :] * wv[l:l + 1]

    def fire_g(j, i):
        pltpu.async_copy(tab_hbm.at[idxr_v.at[j]], rows_v.at[i], gsems[i])

    def fire_s(j, i):
        pltpu.async_copy(rows_v.at[i], acc_sh.at[idxc_v.at[j]], ssems[i],
                         add=True)

    def wait_g(i):
        pltpu.make_async_copy(zeros16_hbm.at[pl.ds(0, CH)], rows_v.at[i],
                              gsems[i]).wait()

    def wait_s(i):
        pltpu.make_async_copy(zeros16_hbm.at[pl.ds(0, CH)], rows_v.at[i],
                              ssems[i]).wait()

    def blk(b, carry):
        chunk0 = (wid * NB + b) * KJ
        pltpu.sync_copy(row_hbm.at[pl.ds(chunk0, KJ)], idxr_v)
        pltpu.sync_copy(col_hbm.at[pl.ds(chunk0, KJ)], idxc_v)
        pltpu.sync_copy(ew_hbm.at[pl.ds(chunk0, KJ)], ew_v)

        fire_g(0, 0)
        for j in range(3):
            fire_g(j + 1, j + 1)
            wait_g(j)
            scale(j, j)
            fire_s(j, j)

        def steady(r, carry2):
            for k in range(4):
                j = 3 + r * 4 + k
                bank = (3 + k) % 4
                nb_ = (bank + 1) % 4
                wait_s(nb_)
                fire_g(j + 1, nb_)
                wait_g(bank)
                scale(j, bank)
                fire_s(j, bank)
            return carry2

        lax.fori_loop(0, (KJ - 4) // 4, steady, 0)

        wait_g(3)
        scale(KJ - 1, 3)
        fire_s(KJ - 1, 3)
        for bk in range(4):
            wait_s(bk)
        return carry

    lax.fori_loop(0, NB, blk, 0)
    plsc.subcore_barrier()
    pltpu.sync_copy(acc_sh.at[sl], out_hbm.at[c, sl])


def _d1_body(degp_ref, xT_ref, dinv_ref, xs_ref):
    a = degp_ref[...]
    dinv = lax.rsqrt(1.0 + a[0:1] + a[1:2])
    dinv_ref[...] = dinv
    xs_ref[...] = xT_ref[...] * dinv


def _d2_body(agg1p_ref, xs_ref, dinv_ref, w1T_ref, b1_ref, gsT_ref):
    a = agg1p_ref[...]
    dinv = dinv_ref[...]
    z = dinv * (a[0:3] + a[3:6] + xs_ref[...])
    h1 = jax.nn.relu(
        jnp.dot(w1T_ref[...], z, preferred_element_type=jnp.float32)
        + b1_ref[...])
    gsT_ref[...] = dinv * h1


def _d3_body(agg2p_ref, gs_ref, dinv16_ref, batch_ref, w2_ref, b2_ref,
             wl_ref, bl_ref, out_ref):
    i = pl.program_id(0)
    a = agg2p_ref[...]
    z2 = dinv16_ref[...] * (a[0] + a[1] + gs_ref[...])
    h2 = jax.nn.relu(
        jnp.dot(z2, w2_ref[...], preferred_element_type=jnp.float32)
        + b2_ref[...])
    gid = lax.broadcasted_iota(jnp.int32, (NUM_GRAPHS, NBLK), 0)
    onehot = (gid == batch_ref[0]).astype(jnp.float32)
    pooled = jnp.dot(onehot, h2, preferred_element_type=jnp.float32)
    contrib = jnp.dot(pooled, wl_ref[...], preferred_element_type=jnp.float32)

    @pl.when(i == 0)
    def _():
        out_ref[...] = jnp.broadcast_to(bl_ref[...], (NUM_GRAPHS, 7))

    out_ref[...] += contrib


def kernel(x, edge_index, edge_weight, batch, W1, b1, W2, b2, Wl, bl):
    f32 = jnp.float32
    row = edge_index[0]
    col = edge_index[1]
    pad = E_PAD - N_EDGES
    pad_idx = (jnp.arange(pad, dtype=jnp.int32) * 97) % N_NODES
    row_p = jnp.concatenate([row, pad_idx]).reshape(E_PAD // CH, CH)
    col_p = jnp.concatenate([col, pad_idx]).reshape(E_PAD // CH, CH)
    ew_p = jnp.concatenate([edge_weight, jnp.zeros((pad,), f32)]) \
        .reshape(E_PAD // CH, CH)
    zeros_n = jnp.zeros((N_PAD,), f32)
    zeros_n16 = jnp.zeros((N_PAD, 16), f32)

    deg_parts = _deg_kernel(col_p, ew_p, zeros_n)
    xT = jnp.pad(x, ((0, N_PAD - N_NODES), (0, 0))).T

    dinv, xs = pl.pallas_call(
        _d1_body,
        out_shape=(jax.ShapeDtypeStruct((1, N_PAD), f32),
                   jax.ShapeDtypeStruct((3, N_PAD), f32)),
    )(deg_parts, xT)

    agg1 = _agg1_kernel(row_p, col_p, ew_p, xs[0], xs[1], xs[2], zeros_n)
    gsT = pl.pallas_call(
        _d2_body,
        out_shape=jax.ShapeDtypeStruct((16, N_PAD), f32),
    )(agg1, xs, dinv, W1.T, b1.reshape(16, 1))

    gs = gsT.T

    agg2 = _agg2_kernel(row_p, col_p, ew_p, gs, zeros_n16)

    dinv16 = jnp.broadcast_to(dinv.reshape(N_PAD, 1), (N_PAD, 16))
    batch2d = jnp.pad(batch, (0, N_PAD - N_NODES),
                      constant_values=NUM_GRAPHS).reshape(NGRID, 1, NBLK)

    out = pl.pallas_call(
        _d3_body,
        grid=(NGRID,),
        in_specs=[
            pl.BlockSpec((NC, NBLK, 16), lambda i: (0, i, 0)),
            pl.BlockSpec((NBLK, 16), lambda i: (i, 0)),
            pl.BlockSpec((NBLK, 16), lambda i: (i, 0)),
            pl.BlockSpec((1, 1, NBLK), lambda i: (i, 0, 0)),
            pl.BlockSpec((16, 16), lambda i: (0, 0)),
            pl.BlockSpec((1, 16), lambda i: (0, 0)),
            pl.BlockSpec((16, 7), lambda i: (0, 0)),
            pl.BlockSpec((1, 7), lambda i: (0, 0)),
        ],
        out_specs=pl.BlockSpec((NUM_GRAPHS, 7), lambda i: (0, 0)),
        out_shape=jax.ShapeDtypeStruct((NUM_GRAPHS, 7), f32),
    )(agg2, gs, dinv16, batch2d, W2, b2.reshape(1, 16), Wl, bl.reshape(1, 7))
    return out


# speedup vs baseline: 1.0661x; 1.0661x over previous
import functools

import jax
import jax.numpy as jnp
from jax import lax
from jax.experimental import pallas as pl
from jax.experimental.pallas import tpu as pltpu
from jax.experimental.pallas import tpu_sc as plsc

N_NODES = 100000
N_EDGES = 3200000
NUM_GRAPHS = 64

NC = 2
NS = 16
NW = NC * NS

CH = 128
KJ = 16
SB = CH * KJ
NB = -(-N_EDGES // (NW * SB))
E_PAD = NW * NB * SB

N_PAD = 100352
TSLICE = N_PAD // NS
Q2 = 8
Q1 = 8
NBLK = 2048
NGRID = N_PAD // NBLK

_MESH = plsc.VectorSubcoreMesh(core_axis_name="c", subcore_axis_name="s")


@functools.partial(
    pl.kernel,
    out_type=jax.ShapeDtypeStruct((NC, N_PAD), jnp.float32),
    mesh=_MESH,
    compiler_params=pltpu.CompilerParams(use_tc_tiling_on_sc=False),
    scratch_types=[
        pltpu.VMEM((KJ, CH), jnp.int32),
        pltpu.VMEM((KJ, CH), jnp.float32),
        pltpu.VMEM_SHARED((N_PAD,), jnp.float32),
        pltpu.SemaphoreType.DMA,
    ],
)
def _deg_kernel(col_hbm, ew_hbm, zeros_hbm, out_hbm, idx_v, val_v, acc_sh,
                sem):
    c = lax.axis_index("c")
    s = lax.axis_index("s")
    wid = s * NC + c
    sl = pl.ds(s * TSLICE, TSLICE)
    pltpu.sync_copy(zeros_hbm.at[sl], acc_sh.at[sl])
    plsc.subcore_barrier()

    def blk(b, carry):
        chunk0 = (wid * NB + b) * KJ
        pltpu.sync_copy(col_hbm.at[pl.ds(chunk0, KJ)], idx_v)
        pltpu.sync_copy(ew_hbm.at[pl.ds(chunk0, KJ)], val_v)
        descs = [
            pltpu.async_copy(val_v.at[j], acc_sh.at[idx_v.at[j]], sem,
                             add=True)
            for j in range(KJ)
        ]
        for d in descs:
            d.wait()
        return carry

    lax.fori_loop(0, NB, blk, 0)
    plsc.subcore_barrier()
    pltpu.sync_copy(acc_sh.at[sl], out_hbm.at[c, sl])


@functools.partial(
    pl.kernel,
    out_type=jax.ShapeDtypeStruct((NC * 3, N_PAD), jnp.float32),
    mesh=_MESH,
    compiler_params=pltpu.CompilerParams(use_tc_tiling_on_sc=False),
    scratch_types=[
        pltpu.VMEM((KJ, CH), jnp.int32),
        pltpu.VMEM((KJ, CH), jnp.int32),
        pltpu.VMEM((KJ, CH), jnp.float32),
        pltpu.VMEM((4 * 3, CH), jnp.float32),
        pltpu.VMEM_SHARED((N_PAD,), jnp.float32),
        pltpu.VMEM_SHARED((N_PAD,), jnp.float32),
        pltpu.VMEM_SHARED((N_PAD,), jnp.float32),
        [pltpu.SemaphoreType.DMA] * 4,
        [pltpu.SemaphoreType.DMA] * 4,
    ],
)
def _agg1_kernel(row_hbm, col_hbm, ew_hbm, t0, t1, t2, zeros_hbm, out_hbm,
                 idxr_v, idxc_v, ew_v, gbuf_v, acc0, acc1, acc2, gsems,
                 ssems):
    c = lax.axis_index("c")
    s = lax.axis_index("s")
    wid = s * NC + c
    sl = pl.ds(s * TSLICE, TSLICE)
    accs = [acc0, acc1, acc2]
    tabs = [t0, t1, t2]
    for f in range(3):
        pltpu.sync_copy(zeros_hbm.at[sl], accs[f].at[sl])
    plsc.subcore_barrier()

    def fire_g(j, i):
        for f in range(3):
            pltpu.async_copy(tabs[f].at[idxr_v.at[j]],
                             gbuf_v.at[i * 3 + f], gsems[i])

    def fire_s(j, i):
        for f in range(3):
            pltpu.async_copy(gbuf_v.at[i * 3 + f],
                             accs[f].at[idxc_v.at[j]], ssems[i], add=True)

    def wait_dma(sem):
        for f in range(3):
            pltpu.make_async_copy(t0.at[pl.ds(0, CH)], gbuf_v.at[f],
                                  sem).wait()

    def scale(j, i):
        for f in range(3):
            for g in range(CH // 16):
                qq = pl.ds(g * 16, 16)
                gbuf_v[i * 3 + f, qq] = \
                    gbuf_v[i * 3 + f, qq] * ew_v[j, qq]

    def blk(b, carry):
        chunk0 = (wid * NB + b) * KJ
        pltpu.sync_copy(row_hbm.at[pl.ds(chunk0, KJ)], idxr_v)
        pltpu.sync_copy(col_hbm.at[pl.ds(chunk0, KJ)], idxc_v)
        pltpu.sync_copy(ew_hbm.at[pl.ds(chunk0, KJ)], ew_v)

        fire_g(0, 0)
        for j in range(3):
            fire_g(j + 1, j + 1)
            wait_dma(gsems[j])
            scale(j, j)
            fire_s(j, j)

        def steady(r, carry2):
            for k in range(4):
                j = 3 + r * 4 + k
                bank = (3 + k) % 4
                nb_ = (bank + 1) % 4
                wait_dma(ssems[nb_])
                fire_g(j + 1, nb_)
                wait_dma(gsems[bank])
                scale(j, bank)
                fire_s(j, bank)
            return carry2

        lax.fori_loop(0, (KJ - 4) // 4, steady, 0)

        wait_dma(gsems[3])
        scale(KJ - 1, 3)
        fire_s(KJ - 1, 3)
        for bk in range(4):
            wait_dma(ssems[bk])
        return carry

    lax.fori_loop(0, NB, blk, 0)
    plsc.subcore_barrier()
    for f in range(3):
        pltpu.sync_copy(accs[f].at[sl], out_hbm.at[c * 3 + f, sl])


@functools.partial(
    pl.kernel,
    out_type=jax.ShapeDtypeStruct((NC, N_PAD, 16), jnp.float32),
    mesh=_MESH,
    compiler_params=pltpu.CompilerParams(use_tc_tiling_on_sc=False),
    scratch_types=[
        pltpu.VMEM((KJ, CH), jnp.int32),
        pltpu.VMEM((KJ, CH), jnp.int32),
        pltpu.VMEM((KJ, CH), jnp.float32),
        pltpu.VMEM((4, CH, 16), jnp.float32),
        pltpu.VMEM_SHARED((N_PAD, 16), jnp.float32),
        [pltpu.SemaphoreType.DMA] * 4,
        [pltpu.SemaphoreType.DMA] * 4,
    ],
)
def _agg2_kernel(row_hbm, col_hbm, ew_hbm, tab_hbm, zeros16_hbm, out_hbm,
                 idxr_v, idxc_v, ew_v, rows_v, acc_sh, gsems, ssems):
    c = lax.axis_index("c")
    s = lax.axis_index("s")
    wid = s * NC + c
    sl = pl.ds(s * TSLICE, TSLICE)
    pltpu.sync_copy(zeros16_hbm.at[sl], acc_sh.at[sl])
    plsc.subcore_barrier()

    def scale(j, i):
        for g in range(CH // 16):
            wv = ew_v[j, pl.ds(g * 16, 16)]
            for l in range(16):
                e = g * 16 + l
                rows_v[i, e, :] = rows_v[i, e, :] * wv[l:l + 1]

    def fire_g(j, i):
        pltpu.async_copy(tab_hbm.at[idxr_v.at[j]], rows_v.at[i], gsems[i])

    def fire_s(j, i):
        pltpu.async_copy(rows_v.at[i], acc_sh.at[idxc_v.at[j]], ssems[i],
                         add=True)

    def wait_g(i):
        pltpu.make_async_copy(zeros16_hbm.at[pl.ds(0, CH)], rows_v.at[i],
                              gsems[i]).wait()

    def wait_s(i):
        pltpu.make_async_copy(zeros16_hbm.at[pl.ds(0, CH)], rows_v.at[i],
                              ssems[i]).wait()

    def blk(b, carry):
        chunk0 = (wid * NB + b) * KJ
        pltpu.sync_copy(row_hbm.at[pl.ds(chunk0, KJ)], idxr_v)
        pltpu.sync_copy(col_hbm.at[pl.ds(chunk0, KJ)], idxc_v)
        pltpu.sync_copy(ew_hbm.at[pl.ds(chunk0, KJ)], ew_v)

        fire_g(0, 0)
        for j in range(3):
            fire_g(j + 1, j + 1)
            wait_g(j)
            scale(j, j)
            fire_s(j, j)

        def steady(r, carry2):
            for k in range(4):
                j = 3 + r * 4 + k
                bank = (3 + k) % 4
                nb_ = (bank + 1) % 4
                wait_s(nb_)
                fire_g(j + 1, nb_)
                wait_g(bank)
                scale(j, bank)
                fire_s(j, bank)
            return carry2

        lax.fori_loop(0, (KJ - 4) // 4, steady, 0)

        wait_g(3)
        scale(KJ - 1, 3)
        fire_s(KJ - 1, 3)
        for bk in range(4):
            wait_s(bk)
        return carry

    lax.fori_loop(0, NB, blk, 0)
    plsc.subcore_barrier()
    pltpu.sync_copy(acc_sh.at[sl], out_hbm.at[c, sl])


def _d1_body(degp_ref, xT_ref, dinv_ref, xs_ref):
    a = degp_ref[...]
    dinv = lax.rsqrt(1.0 + a[0:1] + a[1:2])
    dinv_ref[...] = dinv
    xs_ref[...] = xT_ref[...] * dinv


def _d2_body(agg1p_ref, xs_ref, dinv_ref, w1T_ref, b1_ref, gsT_ref):
    a = agg1p_ref[...]
    dinv = dinv_ref[...]
    z = dinv * (a[0:3] + a[3:6] + xs_ref[...])
    h1 = jax.nn.relu(
        jnp.dot(w1T_ref[...], z, preferred_element_type=jnp.float32)
        + b1_ref[...])
    gsT_ref[...] = dinv * h1


def _d3_body(agg2p_ref, gs_ref, degp_ref, batch_ref, w2_ref, b2_ref,
             wl_ref, bl_ref, out_ref):
    i = pl.program_id(0)
    a = agg2p_ref[...]
    d = degp_ref[...]
    dinv = lax.rsqrt(1.0 + d[0:1] + d[1:2])
    z2 = dinv.T * (a[0] + a[1] + gs_ref[...])
    h2 = jax.nn.relu(
        jnp.dot(z2, w2_ref[...], preferred_element_type=jnp.float32)
        + b2_ref[...])
    gid = lax.broadcasted_iota(jnp.int32, (NUM_GRAPHS, NBLK), 0)
    onehot = (gid == batch_ref[0]).astype(jnp.float32)
    pooled = jnp.dot(onehot, h2, preferred_element_type=jnp.float32)
    contrib = jnp.dot(pooled, wl_ref[...], preferred_element_type=jnp.float32)

    @pl.when(i == 0)
    def _():
        out_ref[...] = jnp.broadcast_to(bl_ref[...], (NUM_GRAPHS, 7))

    out_ref[...] += contrib


def kernel(x, edge_index, edge_weight, batch, W1, b1, W2, b2, Wl, bl):
    f32 = jnp.float32
    row = edge_index[0]
    col = edge_index[1]
    pad = E_PAD - N_EDGES
    pad_idx = (jnp.arange(pad, dtype=jnp.int32) * 97) % N_NODES
    row_p = jnp.concatenate([row, pad_idx]).reshape(E_PAD // CH, CH)
    col_p = jnp.concatenate([col, pad_idx]).reshape(E_PAD // CH, CH)
    ew_p = jnp.concatenate([edge_weight, jnp.zeros((pad,), f32)]) \
        .reshape(E_PAD // CH, CH)
    zeros_n = jnp.zeros((N_PAD,), f32)
    zeros_n16 = jnp.zeros((N_PAD, 16), f32)

    deg_parts = _deg_kernel(col_p, ew_p, zeros_n)
    xT = jnp.pad(x, ((0, N_PAD - N_NODES), (0, 0))).T

    dinv, xs = pl.pallas_call(
        _d1_body,
        out_shape=(jax.ShapeDtypeStruct((1, N_PAD), f32),
                   jax.ShapeDtypeStruct((3, N_PAD), f32)),
    )(deg_parts, xT)

    agg1 = _agg1_kernel(row_p, col_p, ew_p, xs[0], xs[1], xs[2], zeros_n)
    gsT = pl.pallas_call(
        _d2_body,
        out_shape=jax.ShapeDtypeStruct((16, N_PAD), f32),
    )(agg1, xs, dinv, W1.T, b1.reshape(16, 1))

    gs = gsT.T

    agg2 = _agg2_kernel(row_p, col_p, ew_p, gs, zeros_n16)

    batch2d = jnp.pad(batch, (0, N_PAD - N_NODES),
                      constant_values=NUM_GRAPHS).reshape(NGRID, 1, NBLK)

    out = pl.pallas_call(
        _d3_body,
        grid=(NGRID,),
        in_specs=[
            pl.BlockSpec((NC, NBLK, 16), lambda i: (0, i, 0)),
            pl.BlockSpec((NBLK, 16), lambda i: (i, 0)),
            pl.BlockSpec((NC, NBLK), lambda i: (0, i)),
            pl.BlockSpec((1, 1, NBLK), lambda i: (i, 0, 0)),
            pl.BlockSpec((16, 16), lambda i: (0, 0)),
            pl.BlockSpec((1, 16), lambda i: (0, 0)),
            pl.BlockSpec((16, 7), lambda i: (0, 0)),
            pl.BlockSpec((1, 7), lambda i: (0, 0)),
        ],
        out_specs=pl.BlockSpec((NUM_GRAPHS, 7), lambda i: (0, 0)),
        out_shape=jax.ShapeDtypeStruct((NUM_GRAPHS, 7), f32),
    )(agg2, gs, deg_parts, batch2d, W2, b2.reshape(1, 16), Wl,
      bl.reshape(1, 7))
    return out
